# Initial kernel scaffold; baseline (speedup 1.0000x reference)
import jax
import jax.numpy as jnp
from jax.experimental import pallas as pl


def kernel(tris):
    raise NotImplementedError("write your pallas kernel here")


# speedup vs baseline: 4.2291x; 4.2291x over previous
import functools

import jax
import jax.numpy as jnp
from jax.experimental import pallas as pl
from jax.experimental.pallas import tpu as pltpu

SZ = 512
NT = 256
BLK = 8


def _raster_kernel(td_ref, linr_ref, linc_ref, ox_ref, oy_ref, oz_ref, oa_ref):
    i = pl.program_id(0)
    px = linr_ref[...]
    py = linc_ref[...]

    ixf = (i * BLK + jax.lax.broadcasted_iota(jnp.int32, (BLK, 1), 0)
           ).astype(jnp.float32)
    iyf = jax.lax.broadcasted_iota(jnp.int32, (1, SZ), 1).astype(jnp.float32)

    zmin = td_ref[14, 0]
    zb0 = jnp.full((BLK, SZ), zmin, dtype=jnp.float32)
    xb0 = jnp.zeros((BLK, SZ), dtype=jnp.float32)
    yb0 = jnp.zeros((BLK, SZ), dtype=jnp.float32)
    ab0 = jnp.zeros((BLK, SZ), dtype=jnp.float32)

    def body(t, carry):
        zb, xb, yb, ab = carry
        v1x = td_ref[0, t]
        v1y = td_ref[1, t]
        v1z = td_ref[2, t]
        v2x = td_ref[3, t]
        v2y = td_ref[4, t]
        v2z = td_ref[5, t]
        v3x = td_ref[6, t]
        v3y = td_ref[7, t]
        v3z = td_ref[8, t]
        invw = td_ref[9, t]
        xminf = td_ref[10, t]
        xmaxf = td_ref[11, t]
        yminf = td_ref[12, t]
        ymaxf = td_ref[13, t]

        pAB = (px - v2x) * (v1y - v2y) - (py - v2y) * (v1x - v2x)
        pCB = (px - v3x) * (v2y - v3y) - (py - v3y) * (v2x - v3x)
        pCA = (px - v1x) * (v3y - v1y) - (py - v1y) * (v3x - v1x)

        m = (jnp.maximum(pAB, 0.0) * jnp.maximum(pCB, 0.0)
             * jnp.maximum(pCA, 0.0)) > 0.0

        w1 = pCB * invw
        w2 = pCA * invw
        w3 = 1.0 - w1 - w2
        z = w1 * v1z + w2 * v2z + w3 * v3z
        xq = w1 * v1x + w2 * v2x + w3 * v3x
        yq = w1 * v1y + w2 * v2y + w3 * v3y

        rowm = (ixf >= xminf) & (ixf < xmaxf)
        colm = (iyf >= yminf) & (iyf < ymaxf)

        cond = m & (z >= zb)
        cond = cond & rowm
        cond = cond & colm

        zb = jnp.where(cond, z, zb)
        xb = jnp.where(cond, xq, xb)
        yb = jnp.where(cond, yq, yb)
        ab = jnp.where(cond, 1.0, ab)
        return zb, xb, yb, ab

    zb, xb, yb, ab = jax.lax.fori_loop(0, NT, body, (zb0, xb0, yb0, ab0))

    ox_ref[...] = xb
    oy_ref[...] = yb
    oz_ref[...] = jnp.where(ab > 0.0, zb, 0.0)
    oa_ref[...] = ab


@functools.partial(jax.jit)
def kernel(tris):
    tris = tris.astype(jnp.float32)
    zmin = tris.reshape(-1, 3).min(axis=0)[-1]
    lin = jnp.linspace(-1.0, 1.0, SZ, dtype=jnp.float32)

    v1 = tris[:, 0, :]
    v2 = tris[:, 1, :]
    v3 = tris[:, 2, :]
    w = (v2[:, 0] - v1[:, 0]) * (v3[:, 1] - v1[:, 1]) - \
        (v2[:, 1] - v1[:, 1]) * (v3[:, 0] - v1[:, 0])
    valid = jnp.logical_not(w < 1e-9)
    invw = 1.0 / jnp.where(valid, w, 1.0)

    tri2d = tris[:, :, :2]
    aabb_min = tri2d.min(axis=1)
    aabb_max = tri2d.max(axis=1)

    def a2i(v):
        return jnp.trunc((jnp.clip(v, -1.0, 1.0) + 1.0) / 2.0 * SZ)

    xminf = a2i(aabb_min[:, 0])
    yminf = a2i(aabb_min[:, 1])
    xmaxf = a2i(aabb_max[:, 0])
    ymaxf = a2i(aabb_max[:, 1])
    xminf = jnp.where(valid, xminf, 0.0)
    xmaxf = jnp.where(valid, xmaxf, 0.0)

    zmin_row = jnp.full((NT,), zmin, dtype=jnp.float32)
    pad = jnp.zeros((NT,), dtype=jnp.float32)
    td = jnp.stack([
        v1[:, 0], v1[:, 1], v1[:, 2],
        v2[:, 0], v2[:, 1], v2[:, 2],
        v3[:, 0], v3[:, 1], v3[:, 2],
        invw, xminf, xmaxf, yminf, ymaxf,
        zmin_row, pad,
    ], axis=0)

    linr = lin[:, None]
    linc = lin[None, :]

    grid = SZ // BLK
    out_sds = jax.ShapeDtypeStruct((SZ, SZ), jnp.float32)
    ox, oy, oz, oa = pl.pallas_call(
        _raster_kernel,
        grid=(grid,),
        in_specs=[
            pl.BlockSpec(memory_space=pltpu.SMEM),
            pl.BlockSpec((BLK, 1), lambda i: (i, 0)),
            pl.BlockSpec((1, SZ), lambda i: (0, 0)),
        ],
        out_specs=[
            pl.BlockSpec((BLK, SZ), lambda i: (i, 0)),
            pl.BlockSpec((BLK, SZ), lambda i: (i, 0)),
            pl.BlockSpec((BLK, SZ), lambda i: (i, 0)),
            pl.BlockSpec((BLK, SZ), lambda i: (i, 0)),
        ],
        out_shape=[out_sds, out_sds, out_sds, out_sds],
    )(td, linr, linc)

    return jnp.stack([ox, oy, oz, oa], axis=-1)


# speedup vs baseline: 8.8277x; 2.0874x over previous
import functools

import jax
import jax.numpy as jnp
from jax.experimental import pallas as pl
from jax.experimental.pallas import tpu as pltpu

SZ = 512
NT = 256
BLK = 16

_NEG_INF = float("-inf")


def _raster_kernel(td_ref, linr_ref, linc_ref, ox_ref, oy_ref, oz_ref, oa_ref):
    i = pl.program_id(0)
    px = linr_ref[...]
    py = linc_ref[...]

    ixf = (i * BLK + jax.lax.broadcasted_iota(jnp.int32, (BLK, 1), 0)
           ).astype(jnp.float32)
    iyf = jax.lax.broadcasted_iota(jnp.int32, (1, SZ), 1).astype(jnp.float32)

    zmin = td_ref[13, 0]
    zb0 = jnp.full((BLK, SZ), zmin, dtype=jnp.float32)
    ab0 = jnp.zeros((BLK, SZ), dtype=jnp.float32)

    def body(t, carry):
        zb, ab = carry
        v1x = td_ref[0, t]
        v1y = td_ref[1, t]
        v2x = td_ref[2, t]
        v2y = td_ref[3, t]
        v3x = td_ref[4, t]
        v3y = td_ref[5, t]
        az = td_ref[6, t]
        bz = td_ref[7, t]
        cz = td_ref[8, t]
        xminf = td_ref[9, t]
        xmaxf = td_ref[10, t]
        yminf = td_ref[11, t]
        ymaxf = td_ref[12, t]

        pAB = (px - v2x) * (v1y - v2y) - (py - v2y) * (v1x - v2x)
        pCB = (px - v3x) * (v2y - v3y) - (py - v3y) * (v2x - v3x)
        pCA = (px - v1x) * (v3y - v1y) - (py - v1y) * (v3x - v1x)

        m = (jnp.maximum(pAB, 0.0) * jnp.maximum(pCB, 0.0)
             * jnp.maximum(pCA, 0.0)) > 0.0

        rowm = (ixf >= xminf) & (ixf < xmaxf)
        colm = (iyf >= yminf) & (iyf < ymaxf)
        zrow = jnp.where(rowm, bz * px + az, _NEG_INF)
        zcol = jnp.where(colm, cz * py, _NEG_INF)
        z = zrow + zcol

        cond = m & (z >= zb)
        zb = jnp.where(cond, z, zb)
        ab = jnp.where(cond, 1.0, ab)
        return zb, ab

    zb, ab = jax.lax.fori_loop(0, NT, body, (zb0, ab0))

    hit = ab > 0.0
    ox_ref[...] = jnp.where(hit, px, 0.0)
    oy_ref[...] = jnp.where(hit, py, 0.0)
    oz_ref[...] = jnp.where(hit, zb, 0.0)
    oa_ref[...] = ab


@functools.partial(jax.jit)
def kernel(tris):
    tris = tris.astype(jnp.float32)
    zmin = tris.reshape(-1, 3).min(axis=0)[-1]
    lin = jnp.linspace(-1.0, 1.0, SZ, dtype=jnp.float32)

    v1 = tris[:, 0, :]
    v2 = tris[:, 1, :]
    v3 = tris[:, 2, :]
    w = (v2[:, 0] - v1[:, 0]) * (v3[:, 1] - v1[:, 1]) - \
        (v2[:, 1] - v1[:, 1]) * (v3[:, 0] - v1[:, 0])
    valid = jnp.logical_not(w < 1e-9)
    invw = 1.0 / jnp.where(valid, w, 1.0)

    d1 = v1[:, 2] - v3[:, 2]
    d2 = v2[:, 2] - v3[:, 2]
    bCB = v2[:, 1] - v3[:, 1]
    eCB = v2[:, 0] - v3[:, 0]
    aCB = -v3[:, 0] * bCB + v3[:, 1] * eCB
    bCA = v3[:, 1] - v1[:, 1]
    eCA = v3[:, 0] - v1[:, 0]
    aCA = -v1[:, 0] * bCA + v1[:, 1] * eCA
    az = v3[:, 2] + (aCB * d1 + aCA * d2) * invw
    bz = (bCB * d1 + bCA * d2) * invw
    cz = (-eCB * d1 - eCA * d2) * invw

    tri2d = tris[:, :, :2]
    aabb_min = tri2d.min(axis=1)
    aabb_max = tri2d.max(axis=1)

    def a2i(v):
        return jnp.trunc((jnp.clip(v, -1.0, 1.0) + 1.0) / 2.0 * SZ)

    xminf = a2i(aabb_min[:, 0])
    yminf = a2i(aabb_min[:, 1])
    xmaxf = a2i(aabb_max[:, 0])
    ymaxf = a2i(aabb_max[:, 1])
    xminf = jnp.where(valid, xminf, 0.0)
    xmaxf = jnp.where(valid, xmaxf, 0.0)

    zmin_row = jnp.full((NT,), zmin, dtype=jnp.float32)
    td = jnp.stack([
        v1[:, 0], v1[:, 1], v2[:, 0], v2[:, 1], v3[:, 0], v3[:, 1],
        az, bz, cz,
        xminf, xmaxf, yminf, ymaxf,
        zmin_row,
    ], axis=0)

    linr = lin[:, None]
    linc = lin[None, :]

    grid = SZ // BLK
    out_sds = jax.ShapeDtypeStruct((SZ, SZ), jnp.float32)
    ox, oy, oz, oa = pl.pallas_call(
        _raster_kernel,
        grid=(grid,),
        in_specs=[
            pl.BlockSpec(memory_space=pltpu.SMEM),
            pl.BlockSpec((BLK, 1), lambda i: (i, 0)),
            pl.BlockSpec((1, SZ), lambda i: (0, 0)),
        ],
        out_specs=[
            pl.BlockSpec((BLK, SZ), lambda i: (i, 0)),
            pl.BlockSpec((BLK, SZ), lambda i: (i, 0)),
            pl.BlockSpec((BLK, SZ), lambda i: (i, 0)),
            pl.BlockSpec((BLK, SZ), lambda i: (i, 0)),
        ],
        out_shape=[out_sds, out_sds, out_sds, out_sds],
    )(td, linr, linc)

    return jnp.stack([ox, oy, oz, oa], axis=-1)


# speedup vs baseline: 14.5992x; 1.6538x over previous
import functools

import jax
import jax.numpy as jnp
from jax.experimental import pallas as pl
from jax.experimental.pallas import tpu as pltpu

SZ = 512
NT = 256
BLK = 32

_NEG_INF = float("-inf")


def _raster_kernel(td_ref, linr_ref, linc_ref, ox_ref, oy_ref, oz_ref, oa_ref):
    i = pl.program_id(0)
    px = linr_ref[...]
    py = linc_ref[...]

    ixf = (i * BLK + jax.lax.broadcasted_iota(jnp.int32, (BLK, 1), 0)
           ).astype(jnp.float32)
    iyf = jax.lax.broadcasted_iota(jnp.int32, (1, SZ), 1).astype(jnp.float32)

    zmin = td_ref[13, 0]
    zb0 = jnp.full((BLK, SZ), zmin, dtype=jnp.float32)
    ab0 = jnp.zeros((BLK, SZ), dtype=jnp.float32)

    def body(t, carry):
        zb, ab = carry
        v1x = td_ref[0, t]
        v1y = td_ref[1, t]
        v2x = td_ref[2, t]
        v2y = td_ref[3, t]
        v3x = td_ref[4, t]
        v3y = td_ref[5, t]
        az = td_ref[6, t]
        bz = td_ref[7, t]
        cz = td_ref[8, t]
        xminf = td_ref[9, t]
        xmaxf = td_ref[10, t]
        yminf = td_ref[11, t]
        ymaxf = td_ref[12, t]

        pAB = (px - v2x) * (v1y - v2y) - (py - v2y) * (v1x - v2x)
        pCB = (px - v3x) * (v2y - v3y) - (py - v3y) * (v2x - v3x)
        pCA = (px - v1x) * (v3y - v1y) - (py - v1y) * (v3x - v1x)

        m = (jnp.maximum(pAB, 0.0) * jnp.maximum(pCB, 0.0)
             * jnp.maximum(pCA, 0.0)) > 0.0

        rowm = (ixf >= xminf) & (ixf < xmaxf)
        colm = (iyf >= yminf) & (iyf < ymaxf)
        zrow = jnp.where(rowm, bz * px + az, _NEG_INF)
        zcol = jnp.where(colm, cz * py, _NEG_INF)
        z = zrow + zcol

        cond = m & (z >= zb)
        zb = jnp.where(cond, z, zb)
        ab = jnp.where(cond, 1.0, ab)
        return zb, ab

    zb, ab = jax.lax.fori_loop(0, NT, body, (zb0, ab0))

    hit = ab > 0.0
    ox_ref[...] = jnp.where(hit, px, 0.0)
    oy_ref[...] = jnp.where(hit, py, 0.0)
    oz_ref[...] = jnp.where(hit, zb, 0.0)
    oa_ref[...] = ab


@functools.partial(jax.jit)
def kernel(tris):
    tris = tris.astype(jnp.float32)
    zmin = tris.reshape(-1, 3).min(axis=0)[-1]
    lin = jnp.linspace(-1.0, 1.0, SZ, dtype=jnp.float32)

    v1 = tris[:, 0, :]
    v2 = tris[:, 1, :]
    v3 = tris[:, 2, :]
    w = (v2[:, 0] - v1[:, 0]) * (v3[:, 1] - v1[:, 1]) - \
        (v2[:, 1] - v1[:, 1]) * (v3[:, 0] - v1[:, 0])
    valid = jnp.logical_not(w < 1e-9)
    invw = 1.0 / jnp.where(valid, w, 1.0)

    d1 = v1[:, 2] - v3[:, 2]
    d2 = v2[:, 2] - v3[:, 2]
    bCB = v2[:, 1] - v3[:, 1]
    eCB = v2[:, 0] - v3[:, 0]
    aCB = -v3[:, 0] * bCB + v3[:, 1] * eCB
    bCA = v3[:, 1] - v1[:, 1]
    eCA = v3[:, 0] - v1[:, 0]
    aCA = -v1[:, 0] * bCA + v1[:, 1] * eCA
    az = v3[:, 2] + (aCB * d1 + aCA * d2) * invw
    bz = (bCB * d1 + bCA * d2) * invw
    cz = (-eCB * d1 - eCA * d2) * invw

    tri2d = tris[:, :, :2]
    aabb_min = tri2d.min(axis=1)
    aabb_max = tri2d.max(axis=1)

    def a2i(v):
        return jnp.trunc((jnp.clip(v, -1.0, 1.0) + 1.0) / 2.0 * SZ)

    xminf = a2i(aabb_min[:, 0])
    yminf = a2i(aabb_min[:, 1])
    xmaxf = a2i(aabb_max[:, 0])
    ymaxf = a2i(aabb_max[:, 1])
    xminf = jnp.where(valid, xminf, 0.0)
    xmaxf = jnp.where(valid, xmaxf, 0.0)

    zmin_row = jnp.full((NT,), zmin, dtype=jnp.float32)
    td = jnp.stack([
        v1[:, 0], v1[:, 1], v2[:, 0], v2[:, 1], v3[:, 0], v3[:, 1],
        az, bz, cz,
        xminf, xmaxf, yminf, ymaxf,
        zmin_row,
    ], axis=0)

    linr = lin[:, None]
    linc = lin[None, :]

    grid = SZ // BLK
    out_sds = jax.ShapeDtypeStruct((SZ, SZ), jnp.float32)
    ox, oy, oz, oa = pl.pallas_call(
        _raster_kernel,
        grid=(grid,),
        in_specs=[
            pl.BlockSpec(memory_space=pltpu.SMEM),
            pl.BlockSpec((BLK, 1), lambda i: (i, 0)),
            pl.BlockSpec((1, SZ), lambda i: (0, 0)),
        ],
        out_specs=[
            pl.BlockSpec((BLK, SZ), lambda i: (i, 0)),
            pl.BlockSpec((BLK, SZ), lambda i: (i, 0)),
            pl.BlockSpec((BLK, SZ), lambda i: (i, 0)),
            pl.BlockSpec((BLK, SZ), lambda i: (i, 0)),
        ],
        out_shape=[out_sds, out_sds, out_sds, out_sds],
    )(td, linr, linc)

    return jnp.stack([ox, oy, oz, oa], axis=-1)


# speedup vs baseline: 20.7355x; 1.4203x over previous
import functools

import jax
import jax.numpy as jnp
from jax.experimental import pallas as pl
from jax.experimental.pallas import tpu as pltpu

SZ = 512
NT = 256
BLK = 64

_NEG_INF = float("-inf")


def _raster_kernel(td_ref, linr_ref, linc_ref, ox_ref, oy_ref, oz_ref, oa_ref):
    i = pl.program_id(0)
    px = linr_ref[...]
    py = linc_ref[...]

    ixf = (i * BLK + jax.lax.broadcasted_iota(jnp.int32, (BLK, 1), 0)
           ).astype(jnp.float32)
    iyf = jax.lax.broadcasted_iota(jnp.int32, (1, SZ), 1).astype(jnp.float32)

    zmin = td_ref[13, 0]
    zb0 = jnp.full((BLK, SZ), zmin, dtype=jnp.float32)
    ab0 = jnp.zeros((BLK, SZ), dtype=jnp.float32)

    def body(t, carry):
        zb, ab = carry
        v1x = td_ref[0, t]
        v1y = td_ref[1, t]
        v2x = td_ref[2, t]
        v2y = td_ref[3, t]
        v3x = td_ref[4, t]
        v3y = td_ref[5, t]
        az = td_ref[6, t]
        bz = td_ref[7, t]
        cz = td_ref[8, t]
        xminf = td_ref[9, t]
        xmaxf = td_ref[10, t]
        yminf = td_ref[11, t]
        ymaxf = td_ref[12, t]

        pAB = (px - v2x) * (v1y - v2y) - (py - v2y) * (v1x - v2x)
        pCB = (px - v3x) * (v2y - v3y) - (py - v3y) * (v2x - v3x)
        pCA = (px - v1x) * (v3y - v1y) - (py - v1y) * (v3x - v1x)

        m = (jnp.maximum(pAB, 0.0) * jnp.maximum(pCB, 0.0)
             * jnp.maximum(pCA, 0.0)) > 0.0

        rowm = (ixf >= xminf) & (ixf < xmaxf)
        colm = (iyf >= yminf) & (iyf < ymaxf)
        zrow = jnp.where(rowm, bz * px + az, _NEG_INF)
        zcol = jnp.where(colm, cz * py, _NEG_INF)
        z = zrow + zcol

        cond = m & (z >= zb)
        zb = jnp.where(cond, z, zb)
        ab = jnp.where(cond, 1.0, ab)
        return zb, ab

    zb, ab = jax.lax.fori_loop(0, NT, body, (zb0, ab0))

    hit = ab > 0.0
    ox_ref[...] = jnp.where(hit, px, 0.0)
    oy_ref[...] = jnp.where(hit, py, 0.0)
    oz_ref[...] = jnp.where(hit, zb, 0.0)
    oa_ref[...] = ab


@functools.partial(jax.jit)
def kernel(tris):
    tris = tris.astype(jnp.float32)
    zmin = tris.reshape(-1, 3).min(axis=0)[-1]
    lin = jnp.linspace(-1.0, 1.0, SZ, dtype=jnp.float32)

    v1 = tris[:, 0, :]
    v2 = tris[:, 1, :]
    v3 = tris[:, 2, :]
    w = (v2[:, 0] - v1[:, 0]) * (v3[:, 1] - v1[:, 1]) - \
        (v2[:, 1] - v1[:, 1]) * (v3[:, 0] - v1[:, 0])
    valid = jnp.logical_not(w < 1e-9)
    invw = 1.0 / jnp.where(valid, w, 1.0)

    d1 = v1[:, 2] - v3[:, 2]
    d2 = v2[:, 2] - v3[:, 2]
    bCB = v2[:, 1] - v3[:, 1]
    eCB = v2[:, 0] - v3[:, 0]
    aCB = -v3[:, 0] * bCB + v3[:, 1] * eCB
    bCA = v3[:, 1] - v1[:, 1]
    eCA = v3[:, 0] - v1[:, 0]
    aCA = -v1[:, 0] * bCA + v1[:, 1] * eCA
    az = v3[:, 2] + (aCB * d1 + aCA * d2) * invw
    bz = (bCB * d1 + bCA * d2) * invw
    cz = (-eCB * d1 - eCA * d2) * invw

    tri2d = tris[:, :, :2]
    aabb_min = tri2d.min(axis=1)
    aabb_max = tri2d.max(axis=1)

    def a2i(v):
        return jnp.trunc((jnp.clip(v, -1.0, 1.0) + 1.0) / 2.0 * SZ)

    xminf = a2i(aabb_min[:, 0])
    yminf = a2i(aabb_min[:, 1])
    xmaxf = a2i(aabb_max[:, 0])
    ymaxf = a2i(aabb_max[:, 1])
    xminf = jnp.where(valid, xminf, 0.0)
    xmaxf = jnp.where(valid, xmaxf, 0.0)

    zmin_row = jnp.full((NT,), zmin, dtype=jnp.float32)
    td = jnp.stack([
        v1[:, 0], v1[:, 1], v2[:, 0], v2[:, 1], v3[:, 0], v3[:, 1],
        az, bz, cz,
        xminf, xmaxf, yminf, ymaxf,
        zmin_row,
    ], axis=0)

    linr = lin[:, None]
    linc = lin[None, :]

    grid = SZ // BLK
    out_sds = jax.ShapeDtypeStruct((SZ, SZ), jnp.float32)
    ox, oy, oz, oa = pl.pallas_call(
        _raster_kernel,
        grid=(grid,),
        in_specs=[
            pl.BlockSpec(memory_space=pltpu.SMEM),
            pl.BlockSpec((BLK, 1), lambda i: (i, 0)),
            pl.BlockSpec((1, SZ), lambda i: (0, 0)),
        ],
        out_specs=[
            pl.BlockSpec((BLK, SZ), lambda i: (i, 0)),
            pl.BlockSpec((BLK, SZ), lambda i: (i, 0)),
            pl.BlockSpec((BLK, SZ), lambda i: (i, 0)),
            pl.BlockSpec((BLK, SZ), lambda i: (i, 0)),
        ],
        out_shape=[out_sds, out_sds, out_sds, out_sds],
    )(td, linr, linc)

    return jnp.stack([ox, oy, oz, oa], axis=-1)


# speedup vs baseline: 25.4481x; 1.2273x over previous
import functools

import jax
import jax.numpy as jnp
from jax.experimental import pallas as pl
from jax.experimental.pallas import tpu as pltpu

SZ = 512
NT = 256
BLK = 128

_NEG_INF = float("-inf")


def _raster_kernel(td_ref, linr_ref, linc_ref, ox_ref, oy_ref, oz_ref, oa_ref):
    i = pl.program_id(0)
    px = linr_ref[...]
    py = linc_ref[...]

    ixf = (i * BLK + jax.lax.broadcasted_iota(jnp.int32, (BLK, 1), 0)
           ).astype(jnp.float32)
    iyf = jax.lax.broadcasted_iota(jnp.int32, (1, SZ), 1).astype(jnp.float32)

    zmin = td_ref[13, 0]
    zb0 = jnp.full((BLK, SZ), zmin, dtype=jnp.float32)
    ab0 = jnp.zeros((BLK, SZ), dtype=jnp.float32)

    def body(t, carry):
        zb, ab = carry
        v1x = td_ref[0, t]
        v1y = td_ref[1, t]
        v2x = td_ref[2, t]
        v2y = td_ref[3, t]
        v3x = td_ref[4, t]
        v3y = td_ref[5, t]
        az = td_ref[6, t]
        bz = td_ref[7, t]
        cz = td_ref[8, t]
        xminf = td_ref[9, t]
        xmaxf = td_ref[10, t]
        yminf = td_ref[11, t]
        ymaxf = td_ref[12, t]

        pAB = (px - v2x) * (v1y - v2y) - (py - v2y) * (v1x - v2x)
        pCB = (px - v3x) * (v2y - v3y) - (py - v3y) * (v2x - v3x)
        pCA = (px - v1x) * (v3y - v1y) - (py - v1y) * (v3x - v1x)

        m = (jnp.maximum(pAB, 0.0) * jnp.maximum(pCB, 0.0)
             * jnp.maximum(pCA, 0.0)) > 0.0

        rowm = (ixf >= xminf) & (ixf < xmaxf)
        colm = (iyf >= yminf) & (iyf < ymaxf)
        zrow = jnp.where(rowm, bz * px + az, _NEG_INF)
        zcol = jnp.where(colm, cz * py, _NEG_INF)
        z = zrow + zcol

        cond = m & (z >= zb)
        zb = jnp.where(cond, z, zb)
        ab = jnp.where(cond, 1.0, ab)
        return zb, ab

    zb, ab = jax.lax.fori_loop(0, NT, body, (zb0, ab0))

    hit = ab > 0.0
    ox_ref[...] = jnp.where(hit, px, 0.0)
    oy_ref[...] = jnp.where(hit, py, 0.0)
    oz_ref[...] = jnp.where(hit, zb, 0.0)
    oa_ref[...] = ab


@functools.partial(jax.jit)
def kernel(tris):
    tris = tris.astype(jnp.float32)
    zmin = tris.reshape(-1, 3).min(axis=0)[-1]
    lin = jnp.linspace(-1.0, 1.0, SZ, dtype=jnp.float32)

    v1 = tris[:, 0, :]
    v2 = tris[:, 1, :]
    v3 = tris[:, 2, :]
    w = (v2[:, 0] - v1[:, 0]) * (v3[:, 1] - v1[:, 1]) - \
        (v2[:, 1] - v1[:, 1]) * (v3[:, 0] - v1[:, 0])
    valid = jnp.logical_not(w < 1e-9)
    invw = 1.0 / jnp.where(valid, w, 1.0)

    d1 = v1[:, 2] - v3[:, 2]
    d2 = v2[:, 2] - v3[:, 2]
    bCB = v2[:, 1] - v3[:, 1]
    eCB = v2[:, 0] - v3[:, 0]
    aCB = -v3[:, 0] * bCB + v3[:, 1] * eCB
    bCA = v3[:, 1] - v1[:, 1]
    eCA = v3[:, 0] - v1[:, 0]
    aCA = -v1[:, 0] * bCA + v1[:, 1] * eCA
    az = v3[:, 2] + (aCB * d1 + aCA * d2) * invw
    bz = (bCB * d1 + bCA * d2) * invw
    cz = (-eCB * d1 - eCA * d2) * invw

    tri2d = tris[:, :, :2]
    aabb_min = tri2d.min(axis=1)
    aabb_max = tri2d.max(axis=1)

    def a2i(v):
        return jnp.trunc((jnp.clip(v, -1.0, 1.0) + 1.0) / 2.0 * SZ)

    xminf = a2i(aabb_min[:, 0])
    yminf = a2i(aabb_min[:, 1])
    xmaxf = a2i(aabb_max[:, 0])
    ymaxf = a2i(aabb_max[:, 1])
    xminf = jnp.where(valid, xminf, 0.0)
    xmaxf = jnp.where(valid, xmaxf, 0.0)

    zmin_row = jnp.full((NT,), zmin, dtype=jnp.float32)
    td = jnp.stack([
        v1[:, 0], v1[:, 1], v2[:, 0], v2[:, 1], v3[:, 0], v3[:, 1],
        az, bz, cz,
        xminf, xmaxf, yminf, ymaxf,
        zmin_row,
    ], axis=0)

    linr = lin[:, None]
    linc = lin[None, :]

    grid = SZ // BLK
    out_sds = jax.ShapeDtypeStruct((SZ, SZ), jnp.float32)
    ox, oy, oz, oa = pl.pallas_call(
        _raster_kernel,
        grid=(grid,),
        in_specs=[
            pl.BlockSpec(memory_space=pltpu.SMEM),
            pl.BlockSpec((BLK, 1), lambda i: (i, 0)),
            pl.BlockSpec((1, SZ), lambda i: (0, 0)),
        ],
        out_specs=[
            pl.BlockSpec((BLK, SZ), lambda i: (i, 0)),
            pl.BlockSpec((BLK, SZ), lambda i: (i, 0)),
            pl.BlockSpec((BLK, SZ), lambda i: (i, 0)),
            pl.BlockSpec((BLK, SZ), lambda i: (i, 0)),
        ],
        out_shape=[out_sds, out_sds, out_sds, out_sds],
    )(td, linr, linc)

    return jnp.stack([ox, oy, oz, oa], axis=-1)


# speedup vs baseline: 26.6504x; 1.0472x over previous
import functools

import jax
import jax.numpy as jnp
from jax.experimental import pallas as pl
from jax.experimental.pallas import tpu as pltpu

SZ = 512
NT = 256
BLK = 256

_NEG_INF = float("-inf")


def _raster_kernel(td_ref, linr_ref, linc_ref, ox_ref, oy_ref, oz_ref, oa_ref):
    i = pl.program_id(0)
    px = linr_ref[...]
    py = linc_ref[...]

    ixf = (i * BLK + jax.lax.broadcasted_iota(jnp.int32, (BLK, 1), 0)
           ).astype(jnp.float32)
    iyf = jax.lax.broadcasted_iota(jnp.int32, (1, SZ), 1).astype(jnp.float32)

    zmin = td_ref[13, 0]
    zb0 = jnp.full((BLK, SZ), zmin, dtype=jnp.float32)
    ab0 = jnp.zeros((BLK, SZ), dtype=jnp.float32)

    def body(t, carry):
        zb, ab = carry
        v1x = td_ref[0, t]
        v1y = td_ref[1, t]
        v2x = td_ref[2, t]
        v2y = td_ref[3, t]
        v3x = td_ref[4, t]
        v3y = td_ref[5, t]
        az = td_ref[6, t]
        bz = td_ref[7, t]
        cz = td_ref[8, t]
        xminf = td_ref[9, t]
        xmaxf = td_ref[10, t]
        yminf = td_ref[11, t]
        ymaxf = td_ref[12, t]

        pAB = (px - v2x) * (v1y - v2y) - (py - v2y) * (v1x - v2x)
        pCB = (px - v3x) * (v2y - v3y) - (py - v3y) * (v2x - v3x)
        pCA = (px - v1x) * (v3y - v1y) - (py - v1y) * (v3x - v1x)

        m = (jnp.maximum(pAB, 0.0) * jnp.maximum(pCB, 0.0)
             * jnp.maximum(pCA, 0.0)) > 0.0

        rowm = (ixf >= xminf) & (ixf < xmaxf)
        colm = (iyf >= yminf) & (iyf < ymaxf)
        zrow = jnp.where(rowm, bz * px + az, _NEG_INF)
        zcol = jnp.where(colm, cz * py, _NEG_INF)
        z = zrow + zcol

        cond = m & (z >= zb)
        zb = jnp.where(cond, z, zb)
        ab = jnp.where(cond, 1.0, ab)
        return zb, ab

    zb, ab = jax.lax.fori_loop(0, NT, body, (zb0, ab0))

    hit = ab > 0.0
    ox_ref[...] = jnp.where(hit, px, 0.0)
    oy_ref[...] = jnp.where(hit, py, 0.0)
    oz_ref[...] = jnp.where(hit, zb, 0.0)
    oa_ref[...] = ab


@functools.partial(jax.jit)
def kernel(tris):
    tris = tris.astype(jnp.float32)
    zmin = tris.reshape(-1, 3).min(axis=0)[-1]
    lin = jnp.linspace(-1.0, 1.0, SZ, dtype=jnp.float32)

    v1 = tris[:, 0, :]
    v2 = tris[:, 1, :]
    v3 = tris[:, 2, :]
    w = (v2[:, 0] - v1[:, 0]) * (v3[:, 1] - v1[:, 1]) - \
        (v2[:, 1] - v1[:, 1]) * (v3[:, 0] - v1[:, 0])
    valid = jnp.logical_not(w < 1e-9)
    invw = 1.0 / jnp.where(valid, w, 1.0)

    d1 = v1[:, 2] - v3[:, 2]
    d2 = v2[:, 2] - v3[:, 2]
    bCB = v2[:, 1] - v3[:, 1]
    eCB = v2[:, 0] - v3[:, 0]
    aCB = -v3[:, 0] * bCB + v3[:, 1] * eCB
    bCA = v3[:, 1] - v1[:, 1]
    eCA = v3[:, 0] - v1[:, 0]
    aCA = -v1[:, 0] * bCA + v1[:, 1] * eCA
    az = v3[:, 2] + (aCB * d1 + aCA * d2) * invw
    bz = (bCB * d1 + bCA * d2) * invw
    cz = (-eCB * d1 - eCA * d2) * invw

    tri2d = tris[:, :, :2]
    aabb_min = tri2d.min(axis=1)
    aabb_max = tri2d.max(axis=1)

    def a2i(v):
        return jnp.trunc((jnp.clip(v, -1.0, 1.0) + 1.0) / 2.0 * SZ)

    xminf = a2i(aabb_min[:, 0])
    yminf = a2i(aabb_min[:, 1])
    xmaxf = a2i(aabb_max[:, 0])
    ymaxf = a2i(aabb_max[:, 1])
    xminf = jnp.where(valid, xminf, 0.0)
    xmaxf = jnp.where(valid, xmaxf, 0.0)

    zmin_row = jnp.full((NT,), zmin, dtype=jnp.float32)
    td = jnp.stack([
        v1[:, 0], v1[:, 1], v2[:, 0], v2[:, 1], v3[:, 0], v3[:, 1],
        az, bz, cz,
        xminf, xmaxf, yminf, ymaxf,
        zmin_row,
    ], axis=0)

    linr = lin[:, None]
    linc = lin[None, :]

    grid = SZ // BLK
    out_sds = jax.ShapeDtypeStruct((SZ, SZ), jnp.float32)
    ox, oy, oz, oa = pl.pallas_call(
        _raster_kernel,
        grid=(grid,),
        in_specs=[
            pl.BlockSpec(memory_space=pltpu.SMEM),
            pl.BlockSpec((BLK, 1), lambda i: (i, 0)),
            pl.BlockSpec((1, SZ), lambda i: (0, 0)),
        ],
        out_specs=[
            pl.BlockSpec((BLK, SZ), lambda i: (i, 0)),
            pl.BlockSpec((BLK, SZ), lambda i: (i, 0)),
            pl.BlockSpec((BLK, SZ), lambda i: (i, 0)),
            pl.BlockSpec((BLK, SZ), lambda i: (i, 0)),
        ],
        out_shape=[out_sds, out_sds, out_sds, out_sds],
    )(td, linr, linc)

    return jnp.stack([ox, oy, oz, oa], axis=-1)


# speedup vs baseline: 26.9839x; 1.0125x over previous
import functools

import jax
import jax.numpy as jnp
from jax.experimental import pallas as pl
from jax.experimental.pallas import tpu as pltpu

SZ = 512
NT = 256
BLK = 512

_NEG_INF = float("-inf")


def _raster_kernel(td_ref, linr_ref, linc_ref, ox_ref, oy_ref, oz_ref, oa_ref):
    i = pl.program_id(0)
    px = linr_ref[...]
    py = linc_ref[...]

    ixf = (i * BLK + jax.lax.broadcasted_iota(jnp.int32, (BLK, 1), 0)
           ).astype(jnp.float32)
    iyf = jax.lax.broadcasted_iota(jnp.int32, (1, SZ), 1).astype(jnp.float32)

    zmin = td_ref[13, 0]
    zb0 = jnp.full((BLK, SZ), zmin, dtype=jnp.float32)
    ab0 = jnp.zeros((BLK, SZ), dtype=jnp.float32)

    def body(t, carry):
        zb, ab = carry
        v1x = td_ref[0, t]
        v1y = td_ref[1, t]
        v2x = td_ref[2, t]
        v2y = td_ref[3, t]
        v3x = td_ref[4, t]
        v3y = td_ref[5, t]
        az = td_ref[6, t]
        bz = td_ref[7, t]
        cz = td_ref[8, t]
        xminf = td_ref[9, t]
        xmaxf = td_ref[10, t]
        yminf = td_ref[11, t]
        ymaxf = td_ref[12, t]

        pAB = (px - v2x) * (v1y - v2y) - (py - v2y) * (v1x - v2x)
        pCB = (px - v3x) * (v2y - v3y) - (py - v3y) * (v2x - v3x)
        pCA = (px - v1x) * (v3y - v1y) - (py - v1y) * (v3x - v1x)

        m = (jnp.maximum(pAB, 0.0) * jnp.maximum(pCB, 0.0)
             * jnp.maximum(pCA, 0.0)) > 0.0

        rowm = (ixf >= xminf) & (ixf < xmaxf)
        colm = (iyf >= yminf) & (iyf < ymaxf)
        zrow = jnp.where(rowm, bz * px + az, _NEG_INF)
        zcol = jnp.where(colm, cz * py, _NEG_INF)
        z = zrow + zcol

        cond = m & (z >= zb)
        zb = jnp.where(cond, z, zb)
        ab = jnp.where(cond, 1.0, ab)
        return zb, ab

    zb, ab = jax.lax.fori_loop(0, NT, body, (zb0, ab0))

    hit = ab > 0.0
    ox_ref[...] = jnp.where(hit, px, 0.0)
    oy_ref[...] = jnp.where(hit, py, 0.0)
    oz_ref[...] = jnp.where(hit, zb, 0.0)
    oa_ref[...] = ab


@functools.partial(jax.jit)
def kernel(tris):
    tris = tris.astype(jnp.float32)
    zmin = tris.reshape(-1, 3).min(axis=0)[-1]
    lin = jnp.linspace(-1.0, 1.0, SZ, dtype=jnp.float32)

    v1 = tris[:, 0, :]
    v2 = tris[:, 1, :]
    v3 = tris[:, 2, :]
    w = (v2[:, 0] - v1[:, 0]) * (v3[:, 1] - v1[:, 1]) - \
        (v2[:, 1] - v1[:, 1]) * (v3[:, 0] - v1[:, 0])
    valid = jnp.logical_not(w < 1e-9)
    invw = 1.0 / jnp.where(valid, w, 1.0)

    d1 = v1[:, 2] - v3[:, 2]
    d2 = v2[:, 2] - v3[:, 2]
    bCB = v2[:, 1] - v3[:, 1]
    eCB = v2[:, 0] - v3[:, 0]
    aCB = -v3[:, 0] * bCB + v3[:, 1] * eCB
    bCA = v3[:, 1] - v1[:, 1]
    eCA = v3[:, 0] - v1[:, 0]
    aCA = -v1[:, 0] * bCA + v1[:, 1] * eCA
    az = v3[:, 2] + (aCB * d1 + aCA * d2) * invw
    bz = (bCB * d1 + bCA * d2) * invw
    cz = (-eCB * d1 - eCA * d2) * invw

    tri2d = tris[:, :, :2]
    aabb_min = tri2d.min(axis=1)
    aabb_max = tri2d.max(axis=1)

    def a2i(v):
        return jnp.trunc((jnp.clip(v, -1.0, 1.0) + 1.0) / 2.0 * SZ)

    xminf = a2i(aabb_min[:, 0])
    yminf = a2i(aabb_min[:, 1])
    xmaxf = a2i(aabb_max[:, 0])
    ymaxf = a2i(aabb_max[:, 1])
    xminf = jnp.where(valid, xminf, 0.0)
    xmaxf = jnp.where(valid, xmaxf, 0.0)

    zmin_row = jnp.full((NT,), zmin, dtype=jnp.float32)
    td = jnp.stack([
        v1[:, 0], v1[:, 1], v2[:, 0], v2[:, 1], v3[:, 0], v3[:, 1],
        az, bz, cz,
        xminf, xmaxf, yminf, ymaxf,
        zmin_row,
    ], axis=0)

    linr = lin[:, None]
    linc = lin[None, :]

    grid = SZ // BLK
    out_sds = jax.ShapeDtypeStruct((SZ, SZ), jnp.float32)
    ox, oy, oz, oa = pl.pallas_call(
        _raster_kernel,
        grid=(grid,),
        in_specs=[
            pl.BlockSpec(memory_space=pltpu.SMEM),
            pl.BlockSpec((BLK, 1), lambda i: (i, 0)),
            pl.BlockSpec((1, SZ), lambda i: (0, 0)),
        ],
        out_specs=[
            pl.BlockSpec((BLK, SZ), lambda i: (i, 0)),
            pl.BlockSpec((BLK, SZ), lambda i: (i, 0)),
            pl.BlockSpec((BLK, SZ), lambda i: (i, 0)),
            pl.BlockSpec((BLK, SZ), lambda i: (i, 0)),
        ],
        out_shape=[out_sds, out_sds, out_sds, out_sds],
    )(td, linr, linc)

    return jnp.stack([ox, oy, oz, oa], axis=-1)


# speedup vs baseline: 37.0988x; 1.3749x over previous
import functools

import jax
import jax.numpy as jnp
from jax.experimental import pallas as pl
from jax.experimental.pallas import tpu as pltpu

SZ = 512
NT = 256
BLK = 512

_NEG_INF = float("-inf")


def _raster_kernel(td_ref, linr_ref, linc_ref, ox_ref, oy_ref, oz_ref, oa_ref):
    i = pl.program_id(0)
    px = linr_ref[...]
    py = linc_ref[...]

    ixf = (i * BLK + jax.lax.broadcasted_iota(jnp.int32, (BLK, 1), 0)
           ).astype(jnp.float32)
    iyf = jax.lax.broadcasted_iota(jnp.int32, (1, SZ), 1).astype(jnp.float32)

    zmin = td_ref[13, 0]
    zb0 = jnp.full((BLK, SZ), zmin, dtype=jnp.float32)

    def body(t, zb):
        v1x = td_ref[0, t]
        v1y = td_ref[1, t]
        v2x = td_ref[2, t]
        v2y = td_ref[3, t]
        v3x = td_ref[4, t]
        v3y = td_ref[5, t]
        az = td_ref[6, t]
        bz = td_ref[7, t]
        cz = td_ref[8, t]
        xminf = td_ref[9, t]
        xmaxf = td_ref[10, t]
        yminf = td_ref[11, t]
        ymaxf = td_ref[12, t]

        pAB = (px - v2x) * (v1y - v2y) - (py - v2y) * (v1x - v2x)
        pCB = (px - v3x) * (v2y - v3y) - (py - v3y) * (v2x - v3x)
        pCA = (px - v1x) * (v3y - v1y) - (py - v1y) * (v3x - v1x)

        emin = jnp.minimum(jnp.minimum(pAB, pCB), pCA)

        rowm = (ixf >= xminf) & (ixf < xmaxf)
        colm = (iyf >= yminf) & (iyf < ymaxf)
        zrow = jnp.where(rowm, bz * px + az, _NEG_INF)
        zcol = jnp.where(colm, cz * py, _NEG_INF)
        z = zrow + zcol

        zcand = jnp.where(emin > 0.0, z, _NEG_INF)
        return jnp.maximum(zb, zcand)

    zb = jax.lax.fori_loop(0, NT, body, zb0)

    hit = zb > zmin
    ox_ref[...] = jnp.where(hit, px, 0.0)
    oy_ref[...] = jnp.where(hit, py, 0.0)
    oz_ref[...] = jnp.where(hit, zb, 0.0)
    oa_ref[...] = jnp.where(hit, 1.0, 0.0)


@functools.partial(jax.jit)
def kernel(tris):
    tris = tris.astype(jnp.float32)
    zmin = tris.reshape(-1, 3).min(axis=0)[-1]
    lin = jnp.linspace(-1.0, 1.0, SZ, dtype=jnp.float32)

    v1 = tris[:, 0, :]
    v2 = tris[:, 1, :]
    v3 = tris[:, 2, :]
    w = (v2[:, 0] - v1[:, 0]) * (v3[:, 1] - v1[:, 1]) - \
        (v2[:, 1] - v1[:, 1]) * (v3[:, 0] - v1[:, 0])
    valid = jnp.logical_not(w < 1e-9)
    invw = 1.0 / jnp.where(valid, w, 1.0)

    d1 = v1[:, 2] - v3[:, 2]
    d2 = v2[:, 2] - v3[:, 2]
    bCB = v2[:, 1] - v3[:, 1]
    eCB = v2[:, 0] - v3[:, 0]
    aCB = -v3[:, 0] * bCB + v3[:, 1] * eCB
    bCA = v3[:, 1] - v1[:, 1]
    eCA = v3[:, 0] - v1[:, 0]
    aCA = -v1[:, 0] * bCA + v1[:, 1] * eCA
    az = v3[:, 2] + (aCB * d1 + aCA * d2) * invw
    bz = (bCB * d1 + bCA * d2) * invw
    cz = (-eCB * d1 - eCA * d2) * invw

    tri2d = tris[:, :, :2]
    aabb_min = tri2d.min(axis=1)
    aabb_max = tri2d.max(axis=1)

    def a2i(v):
        return jnp.trunc((jnp.clip(v, -1.0, 1.0) + 1.0) / 2.0 * SZ)

    xminf = a2i(aabb_min[:, 0])
    yminf = a2i(aabb_min[:, 1])
    xmaxf = a2i(aabb_max[:, 0])
    ymaxf = a2i(aabb_max[:, 1])
    xminf = jnp.where(valid, xminf, 0.0)
    xmaxf = jnp.where(valid, xmaxf, 0.0)

    zmin_row = jnp.full((NT,), zmin, dtype=jnp.float32)
    td = jnp.stack([
        v1[:, 0], v1[:, 1], v2[:, 0], v2[:, 1], v3[:, 0], v3[:, 1],
        az, bz, cz,
        xminf, xmaxf, yminf, ymaxf,
        zmin_row,
    ], axis=0)

    linr = lin[:, None]
    linc = lin[None, :]

    grid = SZ // BLK
    out_sds = jax.ShapeDtypeStruct((SZ, SZ), jnp.float32)
    ox, oy, oz, oa = pl.pallas_call(
        _raster_kernel,
        grid=(grid,),
        in_specs=[
            pl.BlockSpec(memory_space=pltpu.SMEM),
            pl.BlockSpec((BLK, 1), lambda i: (i, 0)),
            pl.BlockSpec((1, SZ), lambda i: (0, 0)),
        ],
        out_specs=[
            pl.BlockSpec((BLK, SZ), lambda i: (i, 0)),
            pl.BlockSpec((BLK, SZ), lambda i: (i, 0)),
            pl.BlockSpec((BLK, SZ), lambda i: (i, 0)),
            pl.BlockSpec((BLK, SZ), lambda i: (i, 0)),
        ],
        out_shape=[out_sds, out_sds, out_sds, out_sds],
    )(td, linr, linc)

    return jnp.stack([ox, oy, oz, oa], axis=-1)


# speedup vs baseline: 52.9419x; 1.4270x over previous
import functools

import jax
import jax.numpy as jnp
from jax.experimental import pallas as pl
from jax.experimental.pallas import tpu as pltpu

SZ = 512
NT = 256
CHR = 16
NCH = SZ // CHR

_NEG_INF = float("-inf")


def _raster_kernel(td_ref, linr_ref, linc_ref,
                   ox_ref, oy_ref, oz_ref, oa_ref, zb_ref):
    py = linc_ref[...]
    iyf = jax.lax.broadcasted_iota(jnp.int32, (1, SZ), 1).astype(jnp.float32)

    zmin = td_ref[13, 0]
    zb_ref[...] = jnp.full((SZ, SZ), zmin, dtype=jnp.float32)

    def body(t, carry):
        v1x = td_ref[0, t]
        v1y = td_ref[1, t]
        v2x = td_ref[2, t]
        v2y = td_ref[3, t]
        v3x = td_ref[4, t]
        v3y = td_ref[5, t]
        az = td_ref[6, t]
        bz = td_ref[7, t]
        cz = td_ref[8, t]
        xminf = td_ref[9, t]
        xmaxf = td_ref[10, t]
        yminf = td_ref[11, t]
        ymaxf = td_ref[12, t]

        tyA = (py - v2y) * (v1x - v2x)
        tyB = (py - v3y) * (v2x - v3x)
        tyC = (py - v1y) * (v3x - v1x)
        colm = (iyf >= yminf) & (iyf < ymaxf)
        zc = jnp.where(colm, cz * py, _NEG_INF)

        for c in range(NCH):
            sl = slice(c * CHR, (c + 1) * CHR)
            pxc = linr_ref[sl, :]
            ixf = (c * CHR
                   + jax.lax.broadcasted_iota(jnp.int32, (CHR, 1), 0)
                   ).astype(jnp.float32)
            txA = (pxc - v2x) * (v1y - v2y)
            txB = (pxc - v3x) * (v2y - v3y)
            txC = (pxc - v1x) * (v3y - v1y)
            rowm = (ixf >= xminf) & (ixf < xmaxf)
            zr = jnp.where(rowm, bz * pxc + az, _NEG_INF)

            pAB = txA - tyA
            pCB = txB - tyB
            pCA = txC - tyC
            emin = jnp.minimum(jnp.minimum(pAB, pCB), pCA)
            z = zr + zc
            zcand = jnp.where(emin > 0.0, z, _NEG_INF)
            zb_ref[sl, :] = jnp.maximum(zb_ref[sl, :], zcand)
        return carry

    jax.lax.fori_loop(0, NT, body, 0)

    zb = zb_ref[...]
    px = linr_ref[...]
    hit = zb > zmin
    ox_ref[...] = jnp.where(hit, px, 0.0)
    oy_ref[...] = jnp.where(hit, py, 0.0)
    oz_ref[...] = jnp.where(hit, zb, 0.0)
    oa_ref[...] = jnp.where(hit, 1.0, 0.0)


@functools.partial(jax.jit)
def kernel(tris):
    tris = tris.astype(jnp.float32)
    zmin = tris.reshape(-1, 3).min(axis=0)[-1]
    lin = jnp.linspace(-1.0, 1.0, SZ, dtype=jnp.float32)

    v1 = tris[:, 0, :]
    v2 = tris[:, 1, :]
    v3 = tris[:, 2, :]
    w = (v2[:, 0] - v1[:, 0]) * (v3[:, 1] - v1[:, 1]) - \
        (v2[:, 1] - v1[:, 1]) * (v3[:, 0] - v1[:, 0])
    valid = jnp.logical_not(w < 1e-9)
    invw = 1.0 / jnp.where(valid, w, 1.0)

    d1 = v1[:, 2] - v3[:, 2]
    d2 = v2[:, 2] - v3[:, 2]
    bCB = v2[:, 1] - v3[:, 1]
    eCB = v2[:, 0] - v3[:, 0]
    aCB = -v3[:, 0] * bCB + v3[:, 1] * eCB
    bCA = v3[:, 1] - v1[:, 1]
    eCA = v3[:, 0] - v1[:, 0]
    aCA = -v1[:, 0] * bCA + v1[:, 1] * eCA
    az = v3[:, 2] + (aCB * d1 + aCA * d2) * invw
    bz = (bCB * d1 + bCA * d2) * invw
    cz = (-eCB * d1 - eCA * d2) * invw

    tri2d = tris[:, :, :2]
    aabb_min = tri2d.min(axis=1)
    aabb_max = tri2d.max(axis=1)

    def a2i(v):
        return jnp.trunc((jnp.clip(v, -1.0, 1.0) + 1.0) / 2.0 * SZ)

    xminf = a2i(aabb_min[:, 0])
    yminf = a2i(aabb_min[:, 1])
    xmaxf = a2i(aabb_max[:, 0])
    ymaxf = a2i(aabb_max[:, 1])
    xminf = jnp.where(valid, xminf, 0.0)
    xmaxf = jnp.where(valid, xmaxf, 0.0)

    zmin_row = jnp.full((NT,), zmin, dtype=jnp.float32)
    td = jnp.stack([
        v1[:, 0], v1[:, 1], v2[:, 0], v2[:, 1], v3[:, 0], v3[:, 1],
        az, bz, cz,
        xminf, xmaxf, yminf, ymaxf,
        zmin_row,
    ], axis=0)

    linr = lin[:, None]
    linc = lin[None, :]

    out_sds = jax.ShapeDtypeStruct((SZ, SZ), jnp.float32)
    ox, oy, oz, oa = pl.pallas_call(
        _raster_kernel,
        in_specs=[
            pl.BlockSpec(memory_space=pltpu.SMEM),
            pl.BlockSpec(memory_space=pltpu.VMEM),
            pl.BlockSpec(memory_space=pltpu.VMEM),
        ],
        out_specs=[
            pl.BlockSpec(memory_space=pltpu.VMEM),
            pl.BlockSpec(memory_space=pltpu.VMEM),
            pl.BlockSpec(memory_space=pltpu.VMEM),
            pl.BlockSpec(memory_space=pltpu.VMEM),
        ],
        out_shape=[out_sds, out_sds, out_sds, out_sds],
        scratch_shapes=[pltpu.VMEM((SZ, SZ), jnp.float32)],
    )(td, linr, linc)

    return jnp.stack([ox, oy, oz, oa], axis=-1)
